# Initial kernel scaffold; baseline (speedup 1.0000x reference)
import jax
import jax.numpy as jnp
from jax.experimental import pallas as pl


def kernel(node_features, Wl1, bl1, Wr1, Wl2, bl2, Wr2, fc_w, fc_b, edge_index):
    raise NotImplementedError("write your pallas kernel here")


# speedup vs baseline: 8.7151x; 8.7151x over previous
import functools

import jax
import jax.numpy as jnp
from jax import lax
from jax.experimental import pallas as pl
from jax.experimental.pallas import tpu as pltpu
from jax.experimental.pallas import tpu_sc as plsc

N = 10000
E = 320000
D = 128
DH = D // 2
NC = 2
NS = 16
NW = NC * NS
CH = 128
CPT = 79
CPT1 = NC * CPT
EP = NW * CPT * CH
NP = 10240
RPT = NP // NS
CW = 8


def _sc_mesh():
    return plsc.VectorSubcoreMesh(core_axis_name="c", subcore_axis_name="s")


@functools.partial(
    pl.kernel,
    out_type=(
        jax.ShapeDtypeStruct((NC, NP, DH), jnp.float32),
        jax.ShapeDtypeStruct((NC, NP, CW), jnp.float32),
    ),
    mesh=_sc_mesh(),
    scratch_types=(
        pltpu.VMEM((CPT1, CH), jnp.int32),
        pltpu.VMEM((CPT1, CH), jnp.int32),
        pltpu.VMEM((CH, DH), jnp.float32),
        pltpu.VMEM((CH, CW), jnp.float32),
        pltpu.VMEM_SHARED((NP, DH), jnp.float32),
        pltpu.VMEM_SHARED((NP, CW), jnp.float32),
        pltpu.SemaphoreType.DMA,
    ),
    compiler_params=pltpu.CompilerParams(use_tc_tiling_on_sc=False),
)
def _sc1(xl_hbm, xr_hbm, src_hbm, dst_hbm, zero_hbm, zcw_hbm, ones_hbm,
         agg_out, cnt_out,
         src_v, dst_v, rows_v, ones_v, agg_acc, cnt_acc, sem):
    cid = lax.axis_index("c")
    sid = lax.axis_index("s")
    r0 = sid * RPT

    pltpu.sync_copy(src_hbm.at[sid], src_v)
    pltpu.sync_copy(dst_hbm.at[sid], dst_v)
    pltpu.sync_copy(ones_hbm, ones_v)

    pltpu.sync_copy(zero_hbm, agg_acc.at[pl.ds(r0, RPT), :])
    pltpu.sync_copy(zcw_hbm, cnt_acc.at[pl.ds(r0, RPT), :])
    plsc.subcore_barrier()

    @pl.when(cid == 0)
    def _():
        def chunk(j, carry):
            pltpu.async_copy(xl_hbm.at[src_v.at[j]], rows_v, sem).wait()
            pltpu.sync_copy(rows_v, agg_acc.at[dst_v.at[j]], add=True)
            pltpu.sync_copy(ones_v, cnt_acc.at[dst_v.at[j]], add=True)
            return carry
        lax.fori_loop(0, CPT1, chunk, 0)

    @pl.when(cid == 1)
    def _():
        def chunk(j, carry):
            pltpu.async_copy(xr_hbm.at[src_v.at[j]], rows_v, sem).wait()
            pltpu.sync_copy(rows_v, agg_acc.at[dst_v.at[j]], add=True)
            return carry
        lax.fori_loop(0, CPT1, chunk, 0)

    plsc.subcore_barrier()

    pltpu.sync_copy(agg_acc.at[pl.ds(r0, RPT), :],
                    agg_out.at[cid, pl.ds(r0, RPT), :])
    pltpu.sync_copy(cnt_acc.at[pl.ds(r0, RPT), :],
                    cnt_out.at[cid, pl.ds(r0, RPT), :])


B2 = 512
G2 = NP // B2


def _tc2_body(agg_ref, cnt_ref, x_ref, wl_ref, wr_ref, bl_ref,
              x1_ref, invc_ref):
    i = pl.program_id(0)
    agg = jnp.concatenate([agg_ref[0], agg_ref[1]], axis=1)
    cnt = cnt_ref[0, :, 0:1]
    invc = 1.0 / jnp.maximum(cnt, 1.0)
    row = i * B2 + lax.broadcasted_iota(jnp.int32, (B2, 1), 0)
    valid = row < N
    invc = jnp.where(valid, invc, 0.0)
    mean1 = agg * invc
    h = (jnp.dot(mean1, wl_ref[...], preferred_element_type=jnp.float32)
         + jnp.dot(x_ref[...], wr_ref[...], preferred_element_type=jnp.float32)
         + bl_ref[...])
    x1 = jnp.maximum(h, 0.0)
    x1_ref[...] = jnp.where(valid, x1, 0.0)
    invc_ref[...] = invc[:, 0]


def _tc2(agg_part, cnt_part, xp, wl1t, wr1t, bl1):
    return pl.pallas_call(
        _tc2_body,
        grid=(G2,),
        in_specs=[
            pl.BlockSpec((NC, B2, DH), lambda i: (0, i, 0)),
            pl.BlockSpec((NC, B2, CW), lambda i: (0, i, 0)),
            pl.BlockSpec((B2, D), lambda i: (i, 0)),
            pl.BlockSpec((D, D), lambda i: (0, 0)),
            pl.BlockSpec((D, D), lambda i: (0, 0)),
            pl.BlockSpec((1, D), lambda i: (0, 0)),
        ],
        out_specs=[
            pl.BlockSpec((B2, D), lambda i: (i, 0)),
            pl.BlockSpec((B2,), lambda i: (i,)),
        ],
        out_shape=[
            jax.ShapeDtypeStruct((NP, D), jnp.float32),
            jax.ShapeDtypeStruct((NP,), jnp.float32),
        ],
    )(agg_part, cnt_part, xp, wl1t, wr1t, bl1)


@functools.partial(
    pl.kernel,
    out_type=jax.ShapeDtypeStruct((NC, NP, CW), jnp.float32),
    mesh=_sc_mesh(),
    scratch_types=(
        pltpu.VMEM((NP,), jnp.float32),
        pltpu.VMEM((CPT, CH), jnp.int32),
        pltpu.VMEM((CPT, CH), jnp.int32),
        pltpu.VMEM((CH, CW), jnp.float32),
        pltpu.VMEM_SHARED((NP, CW), jnp.float32),
    ),
    compiler_params=pltpu.CompilerParams(use_tc_tiling_on_sc=False,
                                         needs_layout_passes=False),
)
def _sc3(invc_hbm, src_hbm, dst_hbm, zcw_hbm, zch_hbm,
         a_out,
         invc_v, src_v, dst_v, wrows_v, a_acc):
    cid = lax.axis_index("c")
    sid = lax.axis_index("s")
    wid = sid * NC + cid
    r0 = sid * RPT

    pltpu.sync_copy(invc_hbm, invc_v)
    pltpu.sync_copy(src_hbm.at[wid], src_v)
    pltpu.sync_copy(dst_hbm.at[wid], dst_v)
    pltpu.sync_copy(zch_hbm, wrows_v)
    pltpu.sync_copy(zcw_hbm, a_acc.at[pl.ds(r0, RPT), :])
    plsc.subcore_barrier()

    lane = lax.broadcasted_iota(jnp.int32, (16,), 0)
    col0 = jnp.zeros((16,), jnp.int32)

    def chunk(j, carry):
        for t in range(CH // 16):
            idx_d = dst_v[j, pl.ds(t * 16, 16)]
            w = plsc.load_gather(invc_v, [idx_d])
            plsc.store_scatter(wrows_v, [t * 16 + lane, col0], w)
        pltpu.sync_copy(wrows_v, a_acc.at[src_v.at[j]], add=True)
        return carry

    lax.fori_loop(0, CPT, chunk, 0)
    plsc.subcore_barrier()
    pltpu.sync_copy(a_acc.at[pl.ds(r0, RPT), :],
                    a_out.at[cid, pl.ds(r0, RPT), :])


B4 = 2048
G4 = NP // B4


def _tc4_body(x1_ref, a_ref, wl_ref, wr_ref, fw_ref, bl_ref, fb_ref,
              out_ref, acc):
    i = pl.program_id(0)

    @pl.when(i == 0)
    def _():
        acc[...] = jnp.zeros_like(acc)

    a = (a_ref[0, :, 0:1] + a_ref[1, :, 0:1]) * (1.0 / N)
    ab = jnp.concatenate([a, jnp.full((B4, 1), 1.0 / N, jnp.float32)], axis=1)
    x1b = x1_ref[...]
    part = lax.dot_general(ab, x1b, (((0,), (0,)), ((), ())),
                           preferred_element_type=jnp.float32)
    acc[0:2, :] += part

    @pl.when(i == G4 - 1)
    def _():
        s = acc[0:1, :]
        m1 = acc[1:2, :]
        h = jnp.maximum(
            jnp.dot(s, wl_ref[...], preferred_element_type=jnp.float32)
            + jnp.dot(m1, wr_ref[...], preferred_element_type=jnp.float32)
            + bl_ref[...], 0.0)
        out_ref[...] = (jnp.dot(h, fw_ref[...],
                                preferred_element_type=jnp.float32)
                        + fb_ref[...])


def _tc4(x1, a_part, wl2t, wr2t, fcwt, bl2, fcb):
    return pl.pallas_call(
        _tc4_body,
        grid=(G4,),
        in_specs=[
            pl.BlockSpec((B4, D), lambda i: (i, 0)),
            pl.BlockSpec((NC, B4, CW), lambda i: (0, i, 0)),
            pl.BlockSpec((D, D), lambda i: (0, 0)),
            pl.BlockSpec((D, D), lambda i: (0, 0)),
            pl.BlockSpec((D, D), lambda i: (0, 0)),
            pl.BlockSpec((1, D), lambda i: (0, 0)),
            pl.BlockSpec((1, D), lambda i: (0, 0)),
        ],
        out_specs=pl.BlockSpec((1, D), lambda i: (0, 0)),
        out_shape=jax.ShapeDtypeStruct((1, D), jnp.float32),
        scratch_shapes=[pltpu.VMEM((8, D), jnp.float32)],
    )(x1, a_part, wl2t, wr2t, fcwt, bl2, fcb)


def kernel(node_features, Wl1, bl1, Wr1, Wl2, bl2, Wr2, fc_w, fc_b, edge_index):
    x = node_features.astype(jnp.float32)
    src = edge_index[0].astype(jnp.int32)
    dst = edge_index[1].astype(jnp.int32)

    pad = EP - E
    srcp = jnp.concatenate([src, jnp.zeros((pad,), jnp.int32)])
    dstp = jnp.concatenate([dst, jnp.full((pad,), N, jnp.int32)])
    srcr1 = srcp.reshape(NS, CPT1, CH)
    dstr1 = dstp.reshape(NS, CPT1, CH)
    srcr3 = srcp.reshape(NW, CPT, CH)
    dstr3 = dstp.reshape(NW, CPT, CH)

    xl = x[:, :DH]
    xr = x[:, DH:]
    zero_big = jnp.zeros((RPT, DH), jnp.float32)
    zero_cw = jnp.zeros((RPT, CW), jnp.float32)
    zero_ch = jnp.zeros((CH, CW), jnp.float32)
    ones_rows = jnp.zeros((CH, CW), jnp.float32).at[:, 0].set(1.0)

    agg_part, cnt_part = _sc1(xl, xr, srcr1, dstr1,
                              zero_big, zero_cw, ones_rows)

    xp = jnp.zeros((NP, D), jnp.float32).at[:N, :].set(x)
    x1, invc = _tc2(agg_part, cnt_part, xp,
                    Wl1.T, Wr1.T, bl1.reshape(1, D))

    a_part = _sc3(invc, srcr3, dstr3, zero_cw, zero_ch)

    out = _tc4(x1, a_part, Wl2.T, Wr2.T, fc_w.T,
               bl2.reshape(1, D), fc_b.reshape(1, D))
    return out.reshape(D)
